# baseline (speedup 1.0000x reference)
import functools

import jax
import jax.numpy as jnp
from jax import lax
from jax.experimental import pallas as pl
from jax.experimental.pallas import tpu as pltpu
from jax.experimental.pallas import tpu_sc as plsc

_B = 1024
_R = 50
_D = 10
_DP = 16
_NPAD = 1024
_LANES = 16
_GRID = 4


def _tc_body(x_ref, w_ref, b_ref, cpm_ref, huff_ref, tct_ref,
             y_ref, enc_ref, tc2_ref):
    n_nodes = w_ref.shape[0]
    depth, ncls = cpm_ref.shape
    z = lax.dot_general(x_ref[...], w_ref[...], (((1,), (1,)), ((), ())),
                        preferred_element_type=jnp.float32)
    zb = z + b_ref[...]
    y_ref[:, :n_nodes] = 0.5 * jnp.tanh(0.5 * zb) + 0.5
    @pl.when(pl.program_id(0) == 0)
    def _():
        enc_ref[:depth, :ncls] = cpm_ref[...] + _NPAD * huff_ref[...]
        tc2_ref[...] = lax.transpose(tct_ref[...], (1, 0))


def _tc_stage(x, w2, b_row, cpm_t, huff_t, tc_t):
    blk = _B // _GRID
    return pl.pallas_call(
        _tc_body,
        grid=(_GRID,),
        in_specs=[
            pl.BlockSpec((blk, x.shape[1]), lambda i: (i, 0)),
            pl.BlockSpec(w2.shape, lambda i: (0, 0)),
            pl.BlockSpec(b_row.shape, lambda i: (0, 0)),
            pl.BlockSpec(cpm_t.shape, lambda i: (0, 0)),
            pl.BlockSpec(huff_t.shape, lambda i: (0, 0)),
            pl.BlockSpec(tc_t.shape, lambda i: (0, 0)),
        ],
        out_specs=(
            pl.BlockSpec((blk, _NPAD), lambda i: (i, 0)),
            pl.BlockSpec((_DP, _NPAD), lambda i: (0, 0)),
            pl.BlockSpec((_B, _R), lambda i: (0, 0)),
        ),
        out_shape=(
            jax.ShapeDtypeStruct((_B, _NPAD), jnp.float32),
            jax.ShapeDtypeStruct((_DP, _NPAD), jnp.int32),
            jax.ShapeDtypeStruct((_B, _R), jnp.int32),
        ),
    )(x, w2, b_row, cpm_t, huff_t, tc_t)


def _make_sc_gather(n_cores, n_subcores):
    n_workers = n_cores * n_subcores
    rows_per_w = _B // n_workers
    half = rows_per_w // 2
    mesh = plsc.VectorSubcoreMesh(core_axis_name="c", subcore_axis_name="s")

    @functools.partial(
        pl.kernel,
        mesh=mesh,
        out_type=jax.ShapeDtypeStruct((_B, _R), jnp.float32),
        compiler_params=pltpu.CompilerParams(needs_layout_passes=False),
        scratch_types=[
            pltpu.VMEM((rows_per_w, _NPAD), jnp.float32),
            pltpu.VMEM((_NPAD * _DP,), jnp.int32),
            pltpu.VMEM((rows_per_w, _R), jnp.int32),
            pltpu.VMEM((rows_per_w, _R), jnp.float32),
            pltpu.SemaphoreType.DMA,
            pltpu.SemaphoreType.DMA,
            pltpu.SemaphoreType.DMA,
        ],
    )
    def sc_gather(y_hbm, enc_hbm, tc_hbm, out_hbm,
                  y_v, enc_v, tc_v, out_v, s_tab, s_y0, s_y1):
        wid = lax.axis_index("s") * n_cores + lax.axis_index("c")
        row0 = wid * rows_per_w
        c_y0 = pltpu.async_copy(y_hbm.at[pl.ds(row0, half)],
                                y_v.at[pl.ds(0, half)], s_y0)
        c_y1 = pltpu.async_copy(y_hbm.at[pl.ds(row0 + half, half)],
                                y_v.at[pl.ds(half, half)], s_y1)
        c_enc = pltpu.async_copy(enc_hbm, enc_v, s_tab)
        c_tc = pltpu.async_copy(tc_hbm.at[pl.ds(row0, rows_per_w)], tc_v,
                                s_tab)

        offs = tuple(range(0, _R - _LANES, _LANES)) + (_R - _LANES,)
        lane_iota = lax.iota(jnp.int32, _LANES)

        def row_work(row):
            rowv = jnp.broadcast_to(row, (_LANES,))
            for off in offs:
                col = lane_iota + off
                tc16 = plsc.load_gather(tc_v, [rowv, col])
                prod = None
                for t in range(_D):
                    e = plsc.load_gather(enc_v, [tc16 + t * _NPAD])
                    node = jnp.bitwise_and(e, _NPAD - 1)
                    c = jnp.right_shift(e, 10).astype(jnp.float32)
                    yv = plsc.load_gather(y_v, [rowv, node])
                    f = c + yv - 2.0 * c * yv
                    prod = f if prod is None else prod * f
                plsc.store_scatter(out_v, [rowv, col], prod)

        c_enc.wait()
        c_tc.wait()
        c_y0.wait()

        @plsc.parallel_loop(0, half, 1, unroll=4)
        def row_body0(row):
            row_work(row)

        c_y1.wait()

        @plsc.parallel_loop(half, rows_per_w, 1, unroll=4)
        def row_body1(row):
            row_work(row)

        pltpu.sync_copy(out_v, out_hbm.at[pl.ds(row0, rows_per_w)])

    return sc_gather


def kernel(input_vector, target_classes, W, b, class_path_map, huffman_codes):
    n_nodes = W.shape[0]
    w2 = W[:, :, 0]
    b_row = b.reshape(1, n_nodes)

    y_all, enc_t, tc2 = _tc_stage(input_vector, w2, b_row,
                                  class_path_map.astype(jnp.int32).T,
                                  huffman_codes.astype(jnp.int32).T,
                                  target_classes.astype(jnp.int32).T)

    info = plsc.get_sparse_core_info()
    out = _make_sc_gather(info.num_cores, info.num_subcores)(
        y_all, enc_t.reshape(-1), tc2)
    return out


# speedup vs baseline: 1.0493x; 1.0493x over previous
import functools

import jax
import jax.numpy as jnp
from jax import lax
from jax.experimental import pallas as pl
from jax.experimental.pallas import tpu as pltpu
from jax.experimental.pallas import tpu_sc as plsc

_B = 1024
_R = 50
_D = 10
_DP = 16
_NPAD = 1024
_LANES = 16
_GRID = 4


def _tc_body(x_ref, w_ref, b_ref, cpm_ref, huff_ref, tct_ref,
             y_ref, enc_ref, tc2_ref):
    n_nodes = w_ref.shape[0]
    depth, ncls = cpm_ref.shape
    z = lax.dot_general(x_ref[...], w_ref[...], (((1,), (1,)), ((), ())),
                        preferred_element_type=jnp.float32)
    zb = z + b_ref[...]
    y_ref[:, :n_nodes] = 0.5 * jnp.tanh(0.5 * zb) + 0.5
    e = cpm_ref[...] + _NPAD * huff_ref[...]
    pairs = [e[2 * u:2 * u + 1, :] + 65536 * e[2 * u + 1:2 * u + 2, :]
             for u in range(depth // 2)]
    enc_ref[: depth // 2, :ncls] = jnp.concatenate(pairs, axis=0)
    tc2_ref[...] = lax.transpose(tct_ref[...], (1, 0))


def _tc_stage(x, w2, b_row, cpm_t, huff_t, tc_t):
    return pl.pallas_call(
        _tc_body,
        out_shape=(
            jax.ShapeDtypeStruct((_B, _NPAD), jnp.float32),
            jax.ShapeDtypeStruct((_DP, _NPAD), jnp.int32),
            jax.ShapeDtypeStruct((_B, _R), jnp.int32),
        ),
    )(x, w2, b_row, cpm_t, huff_t, tc_t)


def _make_sc_gather(n_cores, n_subcores):
    n_workers = n_cores * n_subcores
    rows_per_w = _B // n_workers
    half = rows_per_w // 2
    mesh = plsc.VectorSubcoreMesh(core_axis_name="c", subcore_axis_name="s")

    @functools.partial(
        pl.kernel,
        mesh=mesh,
        out_type=jax.ShapeDtypeStruct((_B, _R), jnp.float32),
        compiler_params=pltpu.CompilerParams(needs_layout_passes=False),
        scratch_types=[
            pltpu.VMEM((rows_per_w, _NPAD), jnp.float32),
            pltpu.VMEM((_DP, _NPAD), jnp.int32),
            pltpu.VMEM((rows_per_w, _R), jnp.int32),
            pltpu.VMEM((rows_per_w, _R), jnp.float32),
            pltpu.SemaphoreType.DMA,
            pltpu.SemaphoreType.DMA,
            pltpu.SemaphoreType.DMA,
        ],
    )
    def sc_gather(y_hbm, enc_hbm, tc_hbm, out_hbm,
                  y_v, enc_v, tc_v, out_v, s_tab, s_y0, s_y1):
        wid = lax.axis_index("s") * n_cores + lax.axis_index("c")
        row0 = wid * rows_per_w
        c_y0 = pltpu.async_copy(y_hbm.at[pl.ds(row0, half)],
                                y_v.at[pl.ds(0, half)], s_y0)
        c_y1 = pltpu.async_copy(y_hbm.at[pl.ds(row0 + half, half)],
                                y_v.at[pl.ds(half, half)], s_y1)
        c_enc = pltpu.async_copy(enc_hbm, enc_v, s_tab)
        c_tc = pltpu.async_copy(tc_hbm.at[pl.ds(row0, rows_per_w)], tc_v,
                                s_tab)

        offs = tuple(range(0, _R - _LANES, _LANES)) + (_R - _LANES,)
        lane_iota = lax.iota(jnp.int32, _LANES)

        def row_work(row):
            rowv = jnp.broadcast_to(row, (_LANES,))
            for off in offs:
                col = lane_iota + off
                tc16 = plsc.load_gather(tc_v, [rowv, col])
                prod = None
                for u in range(_D // 2):
                    usplat = jnp.full((_LANES,), u, jnp.int32)
                    e2 = plsc.load_gather(enc_v, [usplat, tc16])
                    for half in (jnp.bitwise_and(e2, 65535),
                                 jnp.right_shift(e2, 16)):
                        node = jnp.bitwise_and(half, _NPAD - 1)
                        c = jnp.right_shift(half, 10).astype(jnp.float32)
                        yv = plsc.load_gather(y_v, [rowv, node])
                        f = c + yv - 2.0 * c * yv
                        prod = f if prod is None else prod * f
                plsc.store_scatter(out_v, [rowv, col], prod)

        c_enc.wait()
        c_tc.wait()
        c_y0.wait()

        @plsc.parallel_loop(0, half, 1, unroll=4)
        def row_body0(row):
            row_work(row)

        c_y1.wait()

        @plsc.parallel_loop(half, rows_per_w, 1, unroll=4)
        def row_body1(row):
            row_work(row)

        pltpu.sync_copy(out_v, out_hbm.at[pl.ds(row0, rows_per_w)])

    return sc_gather


def kernel(input_vector, target_classes, W, b, class_path_map, huffman_codes):
    n_nodes = W.shape[0]
    w2 = W[:, :, 0]
    b_row = b.reshape(1, n_nodes)

    y_all, enc_t, tc2 = _tc_stage(input_vector, w2, b_row,
                                  class_path_map.astype(jnp.int32).T,
                                  huffman_codes.astype(jnp.int32).T,
                                  target_classes.astype(jnp.int32).T)

    info = plsc.get_sparse_core_info()
    out = _make_sc_gather(info.num_cores, info.num_subcores)(
        y_all, enc_t, tc2)
    return out


# speedup vs baseline: 1.0501x; 1.0007x over previous
import functools

import jax
import jax.numpy as jnp
from jax import lax
from jax.experimental import pallas as pl
from jax.experimental.pallas import tpu as pltpu
from jax.experimental.pallas import tpu_sc as plsc

_B = 1024
_R = 50
_D = 10
_DP = 16
_NPAD = 1024
_LANES = 16


def _tc_body(x_ref, w_ref, b_ref, cpm_ref, huff_ref, tct_ref,
             y_ref, enc_ref, tc2_ref):
    n_nodes = w_ref.shape[0]
    depth, ncls = cpm_ref.shape
    z = lax.dot_general(x_ref[...], w_ref[...], (((1,), (1,)), ((), ())),
                        preferred_element_type=jnp.float32)
    zb = z + b_ref[...]
    y_ref[:, :n_nodes] = 0.5 * jnp.tanh(0.5 * zb) + 0.5
    enc_ref[:depth, :ncls] = cpm_ref[...] + _NPAD * huff_ref[...]
    tc2_ref[...] = lax.transpose(tct_ref[...], (1, 0))


def _tc_stage(x, w2, b_row, cpm_t, huff_t, tc_t):
    return pl.pallas_call(
        _tc_body,
        out_shape=(
            jax.ShapeDtypeStruct((_B, _NPAD), jnp.float32),
            jax.ShapeDtypeStruct((_DP, _NPAD), jnp.int32),
            jax.ShapeDtypeStruct((_B, _R), jnp.int32),
        ),
    )(x, w2, b_row, cpm_t, huff_t, tc_t)


def _make_sc_gather(n_cores, n_subcores):
    n_workers = n_cores * n_subcores
    rows_per_w = _B // n_workers
    half = rows_per_w // 2
    mesh = plsc.VectorSubcoreMesh(core_axis_name="c", subcore_axis_name="s")

    @functools.partial(
        pl.kernel,
        mesh=mesh,
        out_type=jax.ShapeDtypeStruct((_B, _R), jnp.float32),
        compiler_params=pltpu.CompilerParams(needs_layout_passes=False),
        scratch_types=[
            pltpu.VMEM((rows_per_w, _NPAD), jnp.float32),
            pltpu.VMEM((_DP, _NPAD), jnp.int32),
            pltpu.VMEM((rows_per_w, _R), jnp.int32),
            pltpu.VMEM((rows_per_w, _R), jnp.float32),
            pltpu.SemaphoreType.DMA,
            pltpu.SemaphoreType.DMA,
            pltpu.SemaphoreType.DMA,
        ],
    )
    def sc_gather(y_hbm, enc_hbm, tc_hbm, out_hbm,
                  y_v, enc_v, tc_v, out_v, s_tab, s_y0, s_y1):
        wid = lax.axis_index("s") * n_cores + lax.axis_index("c")
        row0 = wid * rows_per_w
        c_y0 = pltpu.async_copy(y_hbm.at[pl.ds(row0, half)],
                                y_v.at[pl.ds(0, half)], s_y0)
        c_y1 = pltpu.async_copy(y_hbm.at[pl.ds(row0 + half, half)],
                                y_v.at[pl.ds(half, half)], s_y1)
        c_enc = pltpu.async_copy(enc_hbm, enc_v, s_tab)
        c_tc = pltpu.async_copy(tc_hbm.at[pl.ds(row0, rows_per_w)], tc_v,
                                s_tab)

        offs = tuple(range(0, _R - _LANES, _LANES)) + (_R - _LANES,)
        lane_iota = lax.iota(jnp.int32, _LANES)

        def row_work(row):
            rowv = jnp.broadcast_to(row, (_LANES,))
            for off in offs:
                col = lane_iota + off
                tc16 = plsc.load_gather(tc_v, [rowv, col])
                prod = None
                for t in range(_D):
                    tsplat = jnp.full((_LANES,), t, jnp.int32)
                    e = plsc.load_gather(enc_v, [tsplat, tc16])
                    node = jnp.bitwise_and(e, _NPAD - 1)
                    c = jnp.right_shift(e, 10).astype(jnp.float32)
                    yv = plsc.load_gather(y_v, [rowv, node])
                    f = c + yv - 2.0 * c * yv
                    prod = f if prod is None else prod * f
                plsc.store_scatter(out_v, [rowv, col], prod)

        c_enc.wait()
        c_tc.wait()
        c_y0.wait()

        @plsc.parallel_loop(0, half, 1, unroll=4)
        def row_body0(row):
            row_work(row)

        c_y1.wait()

        @plsc.parallel_loop(half, rows_per_w, 1, unroll=4)
        def row_body1(row):
            row_work(row)

        pltpu.sync_copy(out_v, out_hbm.at[pl.ds(row0, rows_per_w)])

    return sc_gather


def kernel(input_vector, target_classes, W, b, class_path_map, huffman_codes):
    n_nodes = W.shape[0]
    w2 = W[:, :, 0]
    b_row = b.reshape(1, n_nodes)

    y_all, enc_t, tc2 = _tc_stage(input_vector, w2, b_row,
                                  class_path_map.astype(jnp.int32).T,
                                  huffman_codes.astype(jnp.int32).T,
                                  target_classes.astype(jnp.int32).T)

    info = plsc.get_sparse_core_info()
    out = _make_sc_gather(info.num_cores, info.num_subcores)(
        y_all, enc_t, tc2)
    return out
